# baseline (speedup 1.0000x reference)
import numpy as np
import jax
import jax.numpy as jnp
from jax.experimental import pallas as pl
from jax.experimental.pallas import tpu as pltpu

_S = 2048
_K_SIMPLEX = 4
_LEVELS = _K_SIMPLEX + 1
_TAU = 0.25
_ALPHA = 0.5
_BASE = 3
_EPS = 1e-8


def _verify_routing_degeneracy():
    pos = np.arange(_S, dtype=np.float64) / float(_S)
    ks = np.arange(1, _LEVELS + 1, dtype=np.float64)
    scales = np.power(float(_BASE), ks)
    wts = np.power(0.5, ks)
    centers = np.array([0.5, 1.5, 2.5], dtype=np.float64)
    y = np.mod(pos[:, None] * scales[None, :], 3.0)
    d2 = (y[..., None] - centers) ** 2
    z = -d2 / _TAU
    z = z - z.max(axis=-1, keepdims=True)
    p = np.exp(z)
    p /= p.sum(axis=-1, keepdims=True)
    bits = p[..., 2] + _ALPHA * p[..., 1]
    cm = (bits * wts[None, :]).sum(axis=1)

    cm_sorted = np.sort(cm)
    gaps = np.diff(cm_sorted)
    dmax = cm_sorted[-1] - cm_sorted[0]
    if (gaps == 0.0).any():
        raise AssertionError("duplicate Cantor measures: routing not one-hot")
    min_gap_norm = gaps.min() / (dmax + 1e-10)
    if min_gap_norm < 1e-10:
        raise AssertionError(
            f"min normalized Cantor gap {min_gap_norm:.3e} too small; "
            "distance-weight softmax may not be exactly one-hot")


_verify_routing_degeneracy()

_DIM = 1024
_M_TILE = 2048


def _fused_proj_kernel(x_ref, w_ref, b_ref, o_ref):
    x = x_ref[...]
    acc = jnp.dot(x, w_ref[...], preferred_element_type=jnp.float32)
    o_ref[...] = x + acc + b_ref[...]


def kernel(x, W_proj, b_proj):
    B, S, D = x.shape
    x2 = x.reshape(S, D)
    b2 = b_proj.reshape(1, D)
    out = pl.pallas_call(
        _fused_proj_kernel,
        grid=(S // _M_TILE,),
        in_specs=[
            pl.BlockSpec((_M_TILE, D), lambda i: (i, i * 0)),
            pl.BlockSpec((D, D), lambda i: (i * 0, i * 0)),
            pl.BlockSpec((1, D), lambda i: (i * 0, i * 0)),
        ],
        out_specs=pl.BlockSpec((_M_TILE, D), lambda i: (i, i * 0)),
        out_shape=jax.ShapeDtypeStruct((S, D), jnp.float32),
        compiler_params=pltpu.CompilerParams(
            dimension_semantics=("arbitrary",),
        ),
    )(x2, W_proj, b2)
    return out.reshape(B, S, D)


# speedup vs baseline: 1.2006x; 1.2006x over previous
import numpy as np
import jax
import jax.numpy as jnp
from jax.experimental import pallas as pl
from jax.experimental.pallas import tpu as pltpu

_S = 2048
_K_SIMPLEX = 4
_LEVELS = _K_SIMPLEX + 1
_TAU = 0.25
_ALPHA = 0.5
_BASE = 3
_EPS = 1e-8


def _verify_routing_degeneracy():
    pos = np.arange(_S, dtype=np.float64) / float(_S)
    ks = np.arange(1, _LEVELS + 1, dtype=np.float64)
    scales = np.power(float(_BASE), ks)
    wts = np.power(0.5, ks)
    centers = np.array([0.5, 1.5, 2.5], dtype=np.float64)
    y = np.mod(pos[:, None] * scales[None, :], 3.0)
    d2 = (y[..., None] - centers) ** 2
    z = -d2 / _TAU
    z = z - z.max(axis=-1, keepdims=True)
    p = np.exp(z)
    p /= p.sum(axis=-1, keepdims=True)
    bits = p[..., 2] + _ALPHA * p[..., 1]
    cm = (bits * wts[None, :]).sum(axis=1)

    cm_sorted = np.sort(cm)
    gaps = np.diff(cm_sorted)
    dmax = cm_sorted[-1] - cm_sorted[0]
    if (gaps == 0.0).any():
        raise AssertionError("duplicate Cantor measures: routing not one-hot")
    min_gap_norm = gaps.min() / (dmax + 1e-10)
    if min_gap_norm < 1e-10:
        raise AssertionError(
            f"min normalized Cantor gap {min_gap_norm:.3e} too small; "
            "distance-weight softmax may not be exactly one-hot")


_verify_routing_degeneracy()

_DIM = 1024
_M_TILE = 1024


def _fused_proj_kernel(x_ref, w_ref, b_ref, o_ref):
    x = x_ref[...]
    acc = jnp.dot(x, w_ref[...], preferred_element_type=jnp.float32)
    o_ref[...] = x + acc + b_ref[...]


def kernel(x, W_proj, b_proj):
    B, S, D = x.shape
    x2 = x.reshape(S, D)
    b2 = b_proj.reshape(1, D)
    out = pl.pallas_call(
        _fused_proj_kernel,
        grid=(S // _M_TILE,),
        in_specs=[
            pl.BlockSpec((_M_TILE, D), lambda i: (i, i * 0)),
            pl.BlockSpec((D, D), lambda i: (i * 0, i * 0)),
            pl.BlockSpec((1, D), lambda i: (i * 0, i * 0)),
        ],
        out_specs=pl.BlockSpec((_M_TILE, D), lambda i: (i, i * 0)),
        out_shape=jax.ShapeDtypeStruct((S, D), jnp.float32),
        compiler_params=pltpu.CompilerParams(
            dimension_semantics=("parallel",),
        ),
    )(x2, W_proj, b2)
    return out.reshape(B, S, D)
